# baseline (speedup 1.0000x reference)
import numpy as np

import jax
import jax.numpy as jnp
from jax.experimental import pallas as pl
from jax.experimental.pallas import tpu as pltpu

_VMEM_LIMIT = 100 * 1024 * 1024


def _conv0_onehot():
    s = np.zeros((3, 8, 8, 4, 16, 27), np.float32)
    for c in range(3):
        for iy in range(8):
            for ix in range(8):
                for py in range(2):
                    for px in range(2):
                        for oyp in range(4):
                            for oxp in range(4):
                                dy = iy - (2 * oyp + py) + 1
                                dx = ix - (2 * oxp + px) + 1
                                if 0 <= dy <= 2 and 0 <= dx <= 2:
                                    t = dy * 3 + dx
                                    s[c, iy, ix, py * 2 + px, oyp * 4 + oxp,
                                      t * 3 + c] = 1.0
    return s.reshape(192 * 64, 27)


_S0 = _conv0_onehot()


def _fused_vgg_kernel(x_ref, w0_ref, b0_ref, w1a_ref, w1b_ref, b1_ref,
                      w2_ref, b2_ref, f0w_ref, f0b_ref, f1w_ref, f1b_ref,
                      f2w_ref, f2b_ref, o_ref):
    B = x_ref.shape[0]
    f32 = jnp.float32
    bf16 = jnp.bfloat16

    def stream(xs):
        h = jnp.dot(xs.astype(bf16), w0_ref[...], preferred_element_type=f32)
        m = jnp.maximum(jnp.maximum(h[:, 0:2048], h[:, 2048:4096]),
                        jnp.maximum(h[:, 4096:6144], h[:, 6144:8192]))
        a1 = jnp.maximum(m + b0_ref[...], 0.0).astype(bf16)

        oy0 = jnp.dot(a1[:, 0:1024], w1a_ref[:, 0:1024], preferred_element_type=f32)
        oy1 = jnp.dot(a1[:, 0:1536], w1b_ref[:, 0:1024], preferred_element_type=f32)
        oy2 = jnp.dot(a1[:, 512:2048], w1b_ref[:, 1024:2048], preferred_element_type=f32)
        oy3 = jnp.dot(a1[:, 1024:2048], w1a_ref[:, 1024:2048], preferred_element_type=f32)
        r0 = jnp.maximum(jnp.maximum(oy0[:, 0:512], oy0[:, 512:1024]),
                         jnp.maximum(oy1[:, 0:512], oy1[:, 512:1024]))
        r1 = jnp.maximum(jnp.maximum(oy2[:, 0:512], oy2[:, 512:1024]),
                         jnp.maximum(oy3[:, 0:512], oy3[:, 512:1024]))
        a2a = jnp.maximum(r0 + b1_ref[...], 0.0).astype(bf16)
        a2b = jnp.maximum(r1 + b1_ref[...], 0.0).astype(bf16)
        a2 = jnp.concatenate([a2a, a2b], axis=1)

        g = jnp.dot(a2, w2_ref[...], preferred_element_type=f32)
        m2 = jnp.maximum(jnp.maximum(g[:, 0:512], g[:, 512:1024]),
                         jnp.maximum(g[:, 1024:1536], g[:, 1536:2048]))
        h = jnp.maximum(m2 + b2_ref[...], 0.0).astype(bf16)

        h = jnp.dot(h, f0w_ref[...], preferred_element_type=f32)
        h = jnp.maximum(h + f0b_ref[...], 0.0).astype(bf16)
        h = jnp.dot(h, f1w_ref[...], preferred_element_type=f32)
        h = jnp.maximum(h + f1b_ref[...], 0.0).astype(bf16)
        h = jnp.dot(h, f2w_ref[...], preferred_element_type=f32)
        return (h + f2b_ref[...])[:, 0:o_ref.shape[1]]

    Bs = 512 if B % 512 == 0 else B
    for s in range(B // Bs):
        o_ref[s * Bs:(s + 1) * Bs, :] = stream(x_ref[s * Bs:(s + 1) * Bs, :])


def _toeplitz_conv0(w9):
    s0 = jnp.asarray(_S0, jnp.bfloat16)
    w = jnp.dot(s0, w9.reshape(27, 128), preferred_element_type=jnp.float32)
    return w.reshape(192, 8192).astype(jnp.bfloat16)


def _conv1_row_mats(w9):
    z = jnp.zeros((128, 256), w9.dtype)
    ox_of_block = (0, 2, 1, 3)
    bdy = []
    for dy in range(3):
        rows = []
        for x in range(4):
            blocks = []
            for ox in ox_of_block:
                dx = x - ox + 1
                blocks.append(w9[dy * 3 + dx] if 0 <= dx <= 2 else z)
            rows.append(jnp.concatenate(blocks, axis=1))
        bdy.append(jnp.concatenate(rows, axis=0))
    b0, b1, b2 = bdy
    w1_oy0 = jnp.concatenate([b1, b2], axis=0)
    w1_oy1 = jnp.concatenate([b0, b1, b2], axis=0)
    w1_oy2 = jnp.concatenate([b0, b1, b2], axis=0)
    w1_oy3 = jnp.concatenate([b0, b1], axis=0)
    w1a = jnp.concatenate([w1_oy0, w1_oy3], axis=1).astype(jnp.bfloat16)
    w1b = jnp.concatenate([w1_oy1, w1_oy2], axis=1).astype(jnp.bfloat16)
    return w1a, w1b


def _toeplitz_conv2(w9):
    rows = []
    for iy in range(2):
        for ix in range(2):
            blocks = [w9[(iy - oy + 1) * 3 + (ix - ox + 1)]
                      for oy in range(2) for ox in range(2)]
            rows.append(jnp.concatenate(blocks, axis=1))
    return jnp.concatenate(rows, axis=0)


def kernel(x, conv0_w, conv0_b, conv1_w, conv1_b, conv2_w, conv2_b,
           fc0_w, fc0_b, fc1_w, fc1_b, fc2_w, fc2_b):
    n = x.shape[0]
    B = 2048 if n % 2048 == 0 else (128 if n % 128 == 0 else n)

    x2 = x.reshape(n, 192)

    w0 = _toeplitz_conv0(conv0_w)
    b0 = jnp.tile(conv0_b, 16).reshape(1, 2048)
    w1a, w1b = _conv1_row_mats(conv1_w)
    b1 = jnp.tile(conv1_b, 2).reshape(1, 512)
    w2 = _toeplitz_conv2(conv2_w)
    b2 = conv2_b.reshape(1, 512)

    class_num = fc2_w.shape[1]
    npad = 128
    f2w = jnp.pad(fc2_w, ((0, 0), (0, npad - class_num)))
    f2b = jnp.pad(fc2_b, (0, npad - class_num)).reshape(1, npad)

    out = pl.pallas_call(
        _fused_vgg_kernel,
        grid=(n // B,),
        out_shape=jax.ShapeDtypeStruct((n, class_num), jnp.float32),
        in_specs=[
            pl.BlockSpec((B, 192), lambda i: (i, 0)),
            pl.BlockSpec((192, 8192), lambda i: (0, 0)),
            pl.BlockSpec((1, 2048), lambda i: (0, 0)),
            pl.BlockSpec((1024, 2048), lambda i: (0, 0)),
            pl.BlockSpec((1536, 2048), lambda i: (0, 0)),
            pl.BlockSpec((1, 512), lambda i: (0, 0)),
            pl.BlockSpec((1024, 2048), lambda i: (0, 0)),
            pl.BlockSpec((1, 512), lambda i: (0, 0)),
            pl.BlockSpec((512, 512), lambda i: (0, 0)),
            pl.BlockSpec((1, 512), lambda i: (0, 0)),
            pl.BlockSpec((512, 512), lambda i: (0, 0)),
            pl.BlockSpec((1, 512), lambda i: (0, 0)),
            pl.BlockSpec((512, npad), lambda i: (0, 0)),
            pl.BlockSpec((1, npad), lambda i: (0, 0)),
        ],
        out_specs=pl.BlockSpec((B, class_num), lambda i: (i, 0)),
        compiler_params=pltpu.CompilerParams(
            dimension_semantics=("parallel",),
            vmem_limit_bytes=_VMEM_LIMIT,
        ),
    )(x2, w0, b0, w1a, w1b, b1, w2, b2,
      fc0_w, fc0_b.reshape(1, 512), fc1_w, fc1_b.reshape(1, 512), f2w, f2b)
    return out


# speedup vs baseline: 1.2287x; 1.2287x over previous
import numpy as np

import jax
import jax.numpy as jnp
from jax.experimental import pallas as pl
from jax.experimental.pallas import tpu as pltpu

_VMEM_LIMIT = 100 * 1024 * 1024


def _conv0_onehot():
    s = np.zeros((3, 8, 8, 4, 16, 27), np.float32)
    for c in range(3):
        for iy in range(8):
            for ix in range(8):
                for py in range(2):
                    for px in range(2):
                        for oyp in range(4):
                            for oxp in range(4):
                                dy = iy - (2 * oyp + py) + 1
                                dx = ix - (2 * oxp + px) + 1
                                if 0 <= dy <= 2 and 0 <= dx <= 2:
                                    t = dy * 3 + dx
                                    s[c, iy, ix, py * 2 + px, oyp * 4 + oxp,
                                      t * 3 + c] = 1.0
    return s.reshape(192 * 64, 27)


_S0 = _conv0_onehot()


def _fused_vgg_kernel(x_ref, w0_ref, b0_ref, w1a_ref, w1b_ref, b1_ref,
                      w2_ref, b2_ref, f0w_ref, f0b_ref, f1w_ref, f1b_ref,
                      f2w_ref, f2b_ref, o_ref):
    B = x_ref.shape[0]
    f32 = jnp.float32
    bf16 = jnp.bfloat16

    def stream(xs):
        h = jnp.dot(xs.astype(bf16), w0_ref[...], preferred_element_type=f32)
        m = jnp.maximum(jnp.maximum(h[:, 0:2048], h[:, 2048:4096]),
                        jnp.maximum(h[:, 4096:6144], h[:, 6144:8192]))
        b0 = jnp.tile(b0_ref[...], (1, 16))
        a1 = jnp.maximum(m + b0, 0.0).astype(bf16)

        oy0 = jnp.dot(a1[:, 0:1024], w1a_ref[:, 0:1024], preferred_element_type=f32)
        oy1 = jnp.dot(a1[:, 0:1536], w1b_ref[:, 0:1024], preferred_element_type=f32)
        oy2 = jnp.dot(a1[:, 512:2048], w1b_ref[:, 1024:2048], preferred_element_type=f32)
        oy3 = jnp.dot(a1[:, 1024:2048], w1a_ref[:, 1024:2048], preferred_element_type=f32)
        r0 = jnp.maximum(jnp.maximum(oy0[:, 0:512], oy0[:, 512:1024]),
                         jnp.maximum(oy1[:, 0:512], oy1[:, 512:1024]))
        r1 = jnp.maximum(jnp.maximum(oy2[:, 0:512], oy2[:, 512:1024]),
                         jnp.maximum(oy3[:, 0:512], oy3[:, 512:1024]))
        b1 = jnp.tile(b1_ref[...], (1, 2))
        a2a = jnp.maximum(r0 + b1, 0.0).astype(bf16)
        a2b = jnp.maximum(r1 + b1, 0.0).astype(bf16)
        a2 = jnp.concatenate([a2a, a2b], axis=1)

        g = jnp.dot(a2, w2_ref[...], preferred_element_type=f32)
        m2 = jnp.maximum(jnp.maximum(g[:, 0:512], g[:, 512:1024]),
                         jnp.maximum(g[:, 1024:1536], g[:, 1536:2048]))
        h = jnp.maximum(m2 + b2_ref[...], 0.0).astype(bf16)

        h = jnp.dot(h, f0w_ref[...], preferred_element_type=f32)
        h = jnp.maximum(h + f0b_ref[...], 0.0).astype(bf16)
        h = jnp.dot(h, f1w_ref[...], preferred_element_type=f32)
        h = jnp.maximum(h + f1b_ref[...], 0.0).astype(bf16)
        h = jnp.dot(h, f2w_ref[...], preferred_element_type=f32)
        return h + f2b_ref[...]

    Bs = 512 if B % 512 == 0 else B
    for s in range(B // Bs):
        o_ref[s * Bs:(s + 1) * Bs, :] = stream(x_ref[s * Bs:(s + 1) * Bs, :])


def _toeplitz_conv0(w9):
    s0 = jnp.asarray(_S0, jnp.bfloat16)
    w = jnp.dot(s0, w9.reshape(27, 128), preferred_element_type=jnp.float32)
    return w.reshape(192, 8192).astype(jnp.bfloat16)


def _conv1_row_mats(w9):
    z = jnp.zeros((128, 256), w9.dtype)
    ox_of_block = (0, 2, 1, 3)
    bdy = []
    for dy in range(3):
        rows = []
        for x in range(4):
            blocks = []
            for ox in ox_of_block:
                dx = x - ox + 1
                blocks.append(w9[dy * 3 + dx] if 0 <= dx <= 2 else z)
            rows.append(jnp.concatenate(blocks, axis=1))
        bdy.append(jnp.concatenate(rows, axis=0))
    b0, b1, b2 = bdy
    w1_oy0 = jnp.concatenate([b1, b2], axis=0)
    w1_oy1 = jnp.concatenate([b0, b1, b2], axis=0)
    w1_oy2 = jnp.concatenate([b0, b1, b2], axis=0)
    w1_oy3 = jnp.concatenate([b0, b1], axis=0)
    w1a = jnp.concatenate([w1_oy0, w1_oy3], axis=1).astype(jnp.bfloat16)
    w1b = jnp.concatenate([w1_oy1, w1_oy2], axis=1).astype(jnp.bfloat16)
    return w1a, w1b


def _toeplitz_conv2(w9):
    rows = []
    for iy in range(2):
        for ix in range(2):
            blocks = [w9[(iy - oy + 1) * 3 + (ix - ox + 1)]
                      for oy in range(2) for ox in range(2)]
            rows.append(jnp.concatenate(blocks, axis=1))
    return jnp.concatenate(rows, axis=0)


def kernel(x, conv0_w, conv0_b, conv1_w, conv1_b, conv2_w, conv2_b,
           fc0_w, fc0_b, fc1_w, fc1_b, fc2_w, fc2_b):
    n = x.shape[0]
    B = 1024 if n % 1024 == 0 else (128 if n % 128 == 0 else n)

    x2 = x.reshape(n, 192)

    w0 = _toeplitz_conv0(conv0_w)
    w1a, w1b = _conv1_row_mats(conv1_w)
    w2 = _toeplitz_conv2(conv2_w)
    class_num = fc2_w.shape[1]

    out = pl.pallas_call(
        _fused_vgg_kernel,
        grid=(n // B,),
        out_shape=jax.ShapeDtypeStruct((n, class_num), jnp.float32),
        in_specs=[
            pl.BlockSpec((B, 192), lambda i: (i, 0)),
            pl.BlockSpec((192, 8192), lambda i: (0, 0)),
            pl.BlockSpec((1, 128), lambda i: (0, 0)),
            pl.BlockSpec((1024, 2048), lambda i: (0, 0)),
            pl.BlockSpec((1536, 2048), lambda i: (0, 0)),
            pl.BlockSpec((1, 256), lambda i: (0, 0)),
            pl.BlockSpec((1024, 2048), lambda i: (0, 0)),
            pl.BlockSpec((1, 512), lambda i: (0, 0)),
            pl.BlockSpec((512, 512), lambda i: (0, 0)),
            pl.BlockSpec((1, 512), lambda i: (0, 0)),
            pl.BlockSpec((512, 512), lambda i: (0, 0)),
            pl.BlockSpec((1, 512), lambda i: (0, 0)),
            pl.BlockSpec((512, class_num), lambda i: (0, 0)),
            pl.BlockSpec((1, class_num), lambda i: (0, 0)),
        ],
        out_specs=pl.BlockSpec((B, class_num), lambda i: (i, 0)),
        compiler_params=pltpu.CompilerParams(
            dimension_semantics=("parallel",),
            vmem_limit_bytes=_VMEM_LIMIT,
        ),
    )(x2, w0, conv0_b.reshape(1, 128), w1a, w1b, conv1_b.reshape(1, 256),
      w2, conv2_b.reshape(1, 512),
      fc0_w, fc0_b.reshape(1, 512), fc1_w, fc1_b.reshape(1, 512),
      fc2_w, fc2_b.reshape(1, class_num))
    return out


# speedup vs baseline: 1.2437x; 1.0121x over previous
import numpy as np

import jax
import jax.numpy as jnp
from jax.experimental import pallas as pl
from jax.experimental.pallas import tpu as pltpu

_VMEM_LIMIT = 100 * 1024 * 1024


def _conv0_onehot():
    s = np.zeros((3, 8, 8, 4, 16, 27), np.float32)
    for c in range(3):
        for iy in range(8):
            for ix in range(8):
                for py in range(2):
                    for px in range(2):
                        for oyp in range(4):
                            for oxp in range(4):
                                dy = iy - (2 * oyp + py) + 1
                                dx = ix - (2 * oxp + px) + 1
                                if 0 <= dy <= 2 and 0 <= dx <= 2:
                                    t = dy * 3 + dx
                                    s[c, iy, ix, py * 2 + px, oyp * 4 + oxp,
                                      t * 3 + c] = 1.0
    return s.reshape(192 * 64, 27)


_S0 = _conv0_onehot()


def _fused_vgg_kernel(x_ref, w0_ref, b0_ref, w1a_ref, w1b_ref, b1_ref,
                      w2_ref, b2_ref, f0w_ref, f0b_ref, f1w_ref, f1b_ref,
                      f2w_ref, f2b_ref, o_ref):
    B = x_ref.shape[0]
    f32 = jnp.float32
    bf16 = jnp.bfloat16

    def stream(xs):
        h = jnp.dot(xs.astype(bf16), w0_ref[...],
                    preferred_element_type=f32).astype(bf16)
        m = jnp.maximum(jnp.maximum(h[:, 0:2048], h[:, 2048:4096]),
                        jnp.maximum(h[:, 4096:6144], h[:, 6144:8192]))
        b0 = jnp.tile(b0_ref[...], (1, 16))
        a1 = jnp.maximum(m.astype(f32) + b0, 0.0).astype(bf16)

        oy0 = jnp.dot(a1[:, 0:1024], w1a_ref[:, 0:1024], preferred_element_type=f32)
        oy1 = jnp.dot(a1[:, 0:1536], w1b_ref[:, 0:1024], preferred_element_type=f32)
        oy2 = jnp.dot(a1[:, 512:2048], w1b_ref[:, 1024:2048], preferred_element_type=f32)
        oy3 = jnp.dot(a1[:, 1024:2048], w1a_ref[:, 1024:2048], preferred_element_type=f32)
        r0 = jnp.maximum(jnp.maximum(oy0[:, 0:512], oy0[:, 512:1024]),
                         jnp.maximum(oy1[:, 0:512], oy1[:, 512:1024]))
        r1 = jnp.maximum(jnp.maximum(oy2[:, 0:512], oy2[:, 512:1024]),
                         jnp.maximum(oy3[:, 0:512], oy3[:, 512:1024]))
        b1 = jnp.tile(b1_ref[...], (1, 2))
        a2a = jnp.maximum(r0 + b1, 0.0).astype(bf16)
        a2b = jnp.maximum(r1 + b1, 0.0).astype(bf16)
        a2 = jnp.concatenate([a2a, a2b], axis=1)

        g = jnp.dot(a2, w2_ref[...], preferred_element_type=f32)
        m2 = jnp.maximum(jnp.maximum(g[:, 0:512], g[:, 512:1024]),
                         jnp.maximum(g[:, 1024:1536], g[:, 1536:2048]))
        h = jnp.maximum(m2 + b2_ref[...], 0.0).astype(bf16)

        h = jnp.dot(h, f0w_ref[...], preferred_element_type=f32)
        h = jnp.maximum(h + f0b_ref[...], 0.0).astype(bf16)
        h = jnp.dot(h, f1w_ref[...], preferred_element_type=f32)
        h = jnp.maximum(h + f1b_ref[...], 0.0).astype(bf16)
        h = jnp.dot(h, f2w_ref[...], preferred_element_type=f32)
        return h + f2b_ref[...]

    Bs = 512 if B % 512 == 0 else B
    for s in range(B // Bs):
        o_ref[s * Bs:(s + 1) * Bs, :] = stream(x_ref[s * Bs:(s + 1) * Bs, :])


def _toeplitz_conv0(w9):
    s0 = jnp.asarray(_S0, jnp.bfloat16)
    w = jnp.dot(s0, w9.reshape(27, 128), preferred_element_type=jnp.float32)
    return w.reshape(192, 8192).astype(jnp.bfloat16)


def _conv1_row_mats(w9):
    z = jnp.zeros((128, 256), w9.dtype)
    ox_of_block = (0, 2, 1, 3)
    bdy = []
    for dy in range(3):
        rows = []
        for x in range(4):
            blocks = []
            for ox in ox_of_block:
                dx = x - ox + 1
                blocks.append(w9[dy * 3 + dx] if 0 <= dx <= 2 else z)
            rows.append(jnp.concatenate(blocks, axis=1))
        bdy.append(jnp.concatenate(rows, axis=0))
    b0, b1, b2 = bdy
    w1_oy0 = jnp.concatenate([b1, b2], axis=0)
    w1_oy1 = jnp.concatenate([b0, b1, b2], axis=0)
    w1_oy2 = jnp.concatenate([b0, b1, b2], axis=0)
    w1_oy3 = jnp.concatenate([b0, b1], axis=0)
    w1a = jnp.concatenate([w1_oy0, w1_oy3], axis=1).astype(jnp.bfloat16)
    w1b = jnp.concatenate([w1_oy1, w1_oy2], axis=1).astype(jnp.bfloat16)
    return w1a, w1b


def _toeplitz_conv2(w9):
    rows = []
    for iy in range(2):
        for ix in range(2):
            blocks = [w9[(iy - oy + 1) * 3 + (ix - ox + 1)]
                      for oy in range(2) for ox in range(2)]
            rows.append(jnp.concatenate(blocks, axis=1))
    return jnp.concatenate(rows, axis=0)


def kernel(x, conv0_w, conv0_b, conv1_w, conv1_b, conv2_w, conv2_b,
           fc0_w, fc0_b, fc1_w, fc1_b, fc2_w, fc2_b):
    n = x.shape[0]
    B = 1024 if n % 1024 == 0 else (128 if n % 128 == 0 else n)

    x2 = x.reshape(n, 192)

    w0 = _toeplitz_conv0(conv0_w)
    w1a, w1b = _conv1_row_mats(conv1_w)
    w2 = _toeplitz_conv2(conv2_w)
    class_num = fc2_w.shape[1]

    out = pl.pallas_call(
        _fused_vgg_kernel,
        grid=(n // B,),
        out_shape=jax.ShapeDtypeStruct((n, class_num), jnp.float32),
        in_specs=[
            pl.BlockSpec((B, 192), lambda i: (i, 0)),
            pl.BlockSpec((192, 8192), lambda i: (0, 0)),
            pl.BlockSpec((1, 128), lambda i: (0, 0)),
            pl.BlockSpec((1024, 2048), lambda i: (0, 0)),
            pl.BlockSpec((1536, 2048), lambda i: (0, 0)),
            pl.BlockSpec((1, 256), lambda i: (0, 0)),
            pl.BlockSpec((1024, 2048), lambda i: (0, 0)),
            pl.BlockSpec((1, 512), lambda i: (0, 0)),
            pl.BlockSpec((512, 512), lambda i: (0, 0)),
            pl.BlockSpec((1, 512), lambda i: (0, 0)),
            pl.BlockSpec((512, 512), lambda i: (0, 0)),
            pl.BlockSpec((1, 512), lambda i: (0, 0)),
            pl.BlockSpec((512, class_num), lambda i: (0, 0)),
            pl.BlockSpec((1, class_num), lambda i: (0, 0)),
        ],
        out_specs=pl.BlockSpec((B, class_num), lambda i: (i, 0)),
        compiler_params=pltpu.CompilerParams(
            dimension_semantics=("parallel",),
            vmem_limit_bytes=_VMEM_LIMIT,
        ),
    )(x2, w0, conv0_b.reshape(1, 128), w1a, w1b, conv1_b.reshape(1, 256),
      w2, conv2_b.reshape(1, 512),
      fc0_w, fc0_b.reshape(1, 512), fc1_w, fc1_b.reshape(1, 512),
      fc2_w, fc2_b.reshape(1, class_num))
    return out


# speedup vs baseline: 1.2486x; 1.0040x over previous
import numpy as np

import jax
import jax.numpy as jnp
from jax.experimental import pallas as pl
from jax.experimental.pallas import tpu as pltpu

_VMEM_LIMIT = 100 * 1024 * 1024


def _conv0_onehot():
    s = np.zeros((3, 8, 8, 4, 16, 27), np.float32)
    for c in range(3):
        for iy in range(8):
            for ix in range(8):
                for py in range(2):
                    for px in range(2):
                        for oyp in range(4):
                            for oxp in range(4):
                                dy = iy - (2 * oyp + py) + 1
                                dx = ix - (2 * oxp + px) + 1
                                if 0 <= dy <= 2 and 0 <= dx <= 2:
                                    t = dy * 3 + dx
                                    s[c, iy, ix, py * 2 + px, oyp * 4 + oxp,
                                      t * 3 + c] = 1.0
    return s.reshape(192 * 64, 27)


_S0 = _conv0_onehot()


def _fused_vgg_kernel(x_ref, w0_ref, b0_ref, w1a_ref, w1b_ref, b1_ref,
                      w2_ref, b2_ref, f0w_ref, f0b_ref, f1w_ref, f1b_ref,
                      f2w_ref, f2b_ref, o_ref):
    B = x_ref.shape[0]
    f32 = jnp.float32
    bf16 = jnp.bfloat16

    def stream(xs):
        h = jnp.dot(xs.astype(bf16), w0_ref[...],
                    preferred_element_type=f32).astype(bf16)
        m = jnp.maximum(jnp.maximum(h[:, 0:2048], h[:, 2048:4096]),
                        jnp.maximum(h[:, 4096:6144], h[:, 6144:8192]))
        b0 = jnp.tile(b0_ref[...], (1, 16))
        a1 = jnp.maximum(m.astype(f32) + b0, 0.0).astype(bf16)

        oy0 = jnp.dot(a1[:, 0:1024], w1a_ref[:, 0:1024],
                      preferred_element_type=f32).astype(bf16)
        oy1 = jnp.dot(a1[:, 0:1536], w1b_ref[:, 0:1024],
                      preferred_element_type=f32).astype(bf16)
        oy2 = jnp.dot(a1[:, 512:2048], w1b_ref[:, 1024:2048],
                      preferred_element_type=f32).astype(bf16)
        oy3 = jnp.dot(a1[:, 1024:2048], w1a_ref[:, 1024:2048],
                      preferred_element_type=f32).astype(bf16)
        r0 = jnp.maximum(jnp.maximum(oy0[:, 0:512], oy0[:, 512:1024]),
                         jnp.maximum(oy1[:, 0:512], oy1[:, 512:1024]))
        r1 = jnp.maximum(jnp.maximum(oy2[:, 0:512], oy2[:, 512:1024]),
                         jnp.maximum(oy3[:, 0:512], oy3[:, 512:1024]))
        b1 = jnp.tile(b1_ref[...], (1, 2))
        a2a = jnp.maximum(r0.astype(f32) + b1, 0.0).astype(bf16)
        a2b = jnp.maximum(r1.astype(f32) + b1, 0.0).astype(bf16)
        a2 = jnp.concatenate([a2a, a2b], axis=1)

        g = jnp.dot(a2, w2_ref[...],
                    preferred_element_type=f32).astype(bf16)
        m2 = jnp.maximum(jnp.maximum(g[:, 0:512], g[:, 512:1024]),
                         jnp.maximum(g[:, 1024:1536], g[:, 1536:2048]))
        h = jnp.maximum(m2.astype(f32) + b2_ref[...], 0.0).astype(bf16)

        h = jnp.dot(h, f0w_ref[...], preferred_element_type=f32)
        h = jnp.maximum(h + f0b_ref[...], 0.0).astype(bf16)
        h = jnp.dot(h, f1w_ref[...], preferred_element_type=f32)
        h = jnp.maximum(h + f1b_ref[...], 0.0).astype(bf16)
        h = jnp.dot(h, f2w_ref[...], preferred_element_type=f32)
        return h + f2b_ref[...]

    Bs = 512 if B % 512 == 0 else B
    for s in range(B // Bs):
        o_ref[s * Bs:(s + 1) * Bs, :] = stream(x_ref[s * Bs:(s + 1) * Bs, :])


def _toeplitz_conv0(w9):
    s0 = jnp.asarray(_S0, jnp.bfloat16)
    w = jnp.dot(s0, w9.reshape(27, 128), preferred_element_type=jnp.float32)
    return w.reshape(192, 8192).astype(jnp.bfloat16)


def _conv1_row_mats(w9):
    z = jnp.zeros((128, 256), w9.dtype)
    ox_of_block = (0, 2, 1, 3)
    bdy = []
    for dy in range(3):
        rows = []
        for x in range(4):
            blocks = []
            for ox in ox_of_block:
                dx = x - ox + 1
                blocks.append(w9[dy * 3 + dx] if 0 <= dx <= 2 else z)
            rows.append(jnp.concatenate(blocks, axis=1))
        bdy.append(jnp.concatenate(rows, axis=0))
    b0, b1, b2 = bdy
    w1_oy0 = jnp.concatenate([b1, b2], axis=0)
    w1_oy1 = jnp.concatenate([b0, b1, b2], axis=0)
    w1_oy2 = jnp.concatenate([b0, b1, b2], axis=0)
    w1_oy3 = jnp.concatenate([b0, b1], axis=0)
    w1a = jnp.concatenate([w1_oy0, w1_oy3], axis=1).astype(jnp.bfloat16)
    w1b = jnp.concatenate([w1_oy1, w1_oy2], axis=1).astype(jnp.bfloat16)
    return w1a, w1b


def _toeplitz_conv2(w9):
    rows = []
    for iy in range(2):
        for ix in range(2):
            blocks = [w9[(iy - oy + 1) * 3 + (ix - ox + 1)]
                      for oy in range(2) for ox in range(2)]
            rows.append(jnp.concatenate(blocks, axis=1))
    return jnp.concatenate(rows, axis=0)


def kernel(x, conv0_w, conv0_b, conv1_w, conv1_b, conv2_w, conv2_b,
           fc0_w, fc0_b, fc1_w, fc1_b, fc2_w, fc2_b):
    n = x.shape[0]
    B = 1024 if n % 1024 == 0 else (128 if n % 128 == 0 else n)

    x2 = x.reshape(n, 192)

    w0 = _toeplitz_conv0(conv0_w)
    w1a, w1b = _conv1_row_mats(conv1_w)
    w2 = _toeplitz_conv2(conv2_w)
    class_num = fc2_w.shape[1]

    out = pl.pallas_call(
        _fused_vgg_kernel,
        grid=(n // B,),
        out_shape=jax.ShapeDtypeStruct((n, class_num), jnp.float32),
        in_specs=[
            pl.BlockSpec((B, 192), lambda i: (i, 0)),
            pl.BlockSpec((192, 8192), lambda i: (0, 0)),
            pl.BlockSpec((1, 128), lambda i: (0, 0)),
            pl.BlockSpec((1024, 2048), lambda i: (0, 0)),
            pl.BlockSpec((1536, 2048), lambda i: (0, 0)),
            pl.BlockSpec((1, 256), lambda i: (0, 0)),
            pl.BlockSpec((1024, 2048), lambda i: (0, 0)),
            pl.BlockSpec((1, 512), lambda i: (0, 0)),
            pl.BlockSpec((512, 512), lambda i: (0, 0)),
            pl.BlockSpec((1, 512), lambda i: (0, 0)),
            pl.BlockSpec((512, 512), lambda i: (0, 0)),
            pl.BlockSpec((1, 512), lambda i: (0, 0)),
            pl.BlockSpec((512, class_num), lambda i: (0, 0)),
            pl.BlockSpec((1, class_num), lambda i: (0, 0)),
        ],
        out_specs=pl.BlockSpec((B, class_num), lambda i: (i, 0)),
        compiler_params=pltpu.CompilerParams(
            dimension_semantics=("parallel",),
            vmem_limit_bytes=_VMEM_LIMIT,
        ),
    )(x2, w0, conv0_b.reshape(1, 128), w1a, w1b, conv1_b.reshape(1, 256),
      w2, conv2_b.reshape(1, 512),
      fc0_w, fc0_b.reshape(1, 512), fc1_w, fc1_b.reshape(1, 512),
      fc2_w, fc2_b.reshape(1, class_num))
    return out


# speedup vs baseline: 1.2501x; 1.0012x over previous
import numpy as np

import jax
import jax.numpy as jnp
from jax.experimental import pallas as pl
from jax.experimental.pallas import tpu as pltpu

_VMEM_LIMIT = 100 * 1024 * 1024


def _conv0_onehot():
    s = np.zeros((3, 8, 8, 4, 16, 27), np.float32)
    for c in range(3):
        for iy in range(8):
            for ix in range(8):
                for py in range(2):
                    for px in range(2):
                        for oyp in range(4):
                            for oxp in range(4):
                                dy = iy - (2 * oyp + py) + 1
                                dx = ix - (2 * oxp + px) + 1
                                if 0 <= dy <= 2 and 0 <= dx <= 2:
                                    t = dy * 3 + dx
                                    s[c, iy, ix, py * 2 + px, oyp * 4 + oxp,
                                      t * 3 + c] = 1.0
    return s.reshape(192 * 64, 27)


_S0 = _conv0_onehot()


def _fused_vgg_kernel(x_ref, w0_ref, b0_ref, w1a_ref, w1b_ref, b1_ref,
                      w2_ref, b2_ref, f0w_ref, f0b_ref, f1w_ref, f1b_ref,
                      f2w_ref, f2b_ref, o_ref):
    B = x_ref.shape[0]
    f32 = jnp.float32
    bf16 = jnp.bfloat16

    def stream(xs):
        h = jnp.dot(xs.astype(bf16), w0_ref[...],
                    preferred_element_type=f32).astype(bf16)
        m = jnp.maximum(jnp.maximum(h[:, 0:2048], h[:, 2048:4096]),
                        jnp.maximum(h[:, 4096:6144], h[:, 6144:8192]))
        b0 = jnp.tile(b0_ref[...], (1, 16))
        a1 = jnp.maximum(m.astype(f32) + b0, 0.0).astype(bf16)

        oy0 = jnp.dot(a1[:, 0:1024], w1a_ref[:, 0:1024],
                      preferred_element_type=f32).astype(bf16)
        oy1 = jnp.dot(a1[:, 0:1536], w1b_ref[:, 0:1024],
                      preferred_element_type=f32).astype(bf16)
        oy2 = jnp.dot(a1[:, 512:2048], w1b_ref[:, 1024:2048],
                      preferred_element_type=f32).astype(bf16)
        oy3 = jnp.dot(a1[:, 1024:2048], w1a_ref[:, 1024:2048],
                      preferred_element_type=f32).astype(bf16)
        r0 = jnp.maximum(jnp.maximum(oy0[:, 0:512], oy0[:, 512:1024]),
                         jnp.maximum(oy1[:, 0:512], oy1[:, 512:1024]))
        r1 = jnp.maximum(jnp.maximum(oy2[:, 0:512], oy2[:, 512:1024]),
                         jnp.maximum(oy3[:, 0:512], oy3[:, 512:1024]))
        b1 = jnp.tile(b1_ref[...], (1, 2))
        a2a = jnp.maximum(r0.astype(f32) + b1, 0.0).astype(bf16)
        a2b = jnp.maximum(r1.astype(f32) + b1, 0.0).astype(bf16)
        a2 = jnp.concatenate([a2a, a2b], axis=1)

        g = jnp.dot(a2, w2_ref[...],
                    preferred_element_type=f32).astype(bf16)
        m2 = jnp.maximum(jnp.maximum(g[:, 0:512], g[:, 512:1024]),
                         jnp.maximum(g[:, 1024:1536], g[:, 1536:2048]))
        h = jnp.maximum(m2.astype(f32) + b2_ref[...], 0.0).astype(bf16)

        h = jnp.dot(h, f0w_ref[...], preferred_element_type=f32)
        h = jnp.maximum(h + f0b_ref[...], 0.0).astype(bf16)
        h = jnp.dot(h, f1w_ref[...], preferred_element_type=f32)
        h = jnp.maximum(h + f1b_ref[...], 0.0).astype(bf16)
        h = jnp.dot(h, f2w_ref[...], preferred_element_type=f32)
        return h + f2b_ref[...]

    Bs = 512 if B % 512 == 0 else B
    for s in range(B // Bs):
        o_ref[s * Bs:(s + 1) * Bs, :] = stream(x_ref[s * Bs:(s + 1) * Bs, :])


def _toeplitz_conv0(w9):
    s0 = jnp.asarray(_S0, jnp.bfloat16)
    w = jnp.dot(s0, w9.reshape(27, 128), preferred_element_type=jnp.float32)
    return w.reshape(192, 8192).astype(jnp.bfloat16)


def _conv1_row_mats(w9):
    z = jnp.zeros((128, 256), w9.dtype)
    ox_of_block = (0, 2, 1, 3)
    bdy = []
    for dy in range(3):
        rows = []
        for x in range(4):
            blocks = []
            for ox in ox_of_block:
                dx = x - ox + 1
                blocks.append(w9[dy * 3 + dx] if 0 <= dx <= 2 else z)
            rows.append(jnp.concatenate(blocks, axis=1))
        bdy.append(jnp.concatenate(rows, axis=0))
    b0, b1, b2 = bdy
    w1_oy0 = jnp.concatenate([b1, b2], axis=0)
    w1_oy1 = jnp.concatenate([b0, b1, b2], axis=0)
    w1_oy2 = jnp.concatenate([b0, b1, b2], axis=0)
    w1_oy3 = jnp.concatenate([b0, b1], axis=0)
    w1a = jnp.concatenate([w1_oy0, w1_oy3], axis=1).astype(jnp.bfloat16)
    w1b = jnp.concatenate([w1_oy1, w1_oy2], axis=1).astype(jnp.bfloat16)
    return w1a, w1b


def _toeplitz_conv2(w9):
    rows = []
    for iy in range(2):
        for ix in range(2):
            blocks = [w9[(iy - oy + 1) * 3 + (ix - ox + 1)]
                      for oy in range(2) for ox in range(2)]
            rows.append(jnp.concatenate(blocks, axis=1))
    return jnp.concatenate(rows, axis=0)


def kernel(x, conv0_w, conv0_b, conv1_w, conv1_b, conv2_w, conv2_b,
           fc0_w, fc0_b, fc1_w, fc1_b, fc2_w, fc2_b):
    n = x.shape[0]
    B = 1024 if n % 1024 == 0 else (128 if n % 128 == 0 else n)

    x2 = x.reshape(n, 192)

    w0 = _toeplitz_conv0(conv0_w)
    w1a, w1b = _conv1_row_mats(conv1_w)
    w2 = _toeplitz_conv2(conv2_w)
    class_num = fc2_w.shape[1]

    out = pl.pallas_call(
        _fused_vgg_kernel,
        grid=(n // B,),
        out_shape=jax.ShapeDtypeStruct((n, class_num), jnp.float32),
        in_specs=[
            pl.BlockSpec((B, 192), lambda i: (i, 0)),
            pl.BlockSpec((192, 8192), lambda i: (0, 0)),
            pl.BlockSpec((1, 128), lambda i: (0, 0)),
            pl.BlockSpec((1024, 2048), lambda i: (0, 0)),
            pl.BlockSpec((1536, 2048), lambda i: (0, 0)),
            pl.BlockSpec((1, 256), lambda i: (0, 0)),
            pl.BlockSpec((1024, 2048), lambda i: (0, 0)),
            pl.BlockSpec((1, 512), lambda i: (0, 0)),
            pl.BlockSpec((512, 512), lambda i: (0, 0)),
            pl.BlockSpec((1, 512), lambda i: (0, 0)),
            pl.BlockSpec((512, 512), lambda i: (0, 0)),
            pl.BlockSpec((1, 512), lambda i: (0, 0)),
            pl.BlockSpec((512, class_num), lambda i: (0, 0)),
            pl.BlockSpec((1, class_num), lambda i: (0, 0)),
        ],
        out_specs=pl.BlockSpec((B, class_num), lambda i: (i, 0)),
        compiler_params=pltpu.CompilerParams(
            dimension_semantics=("arbitrary",),
            vmem_limit_bytes=_VMEM_LIMIT,
        ),
    )(x2, w0, conv0_b.reshape(1, 128), w1a, w1b, conv1_b.reshape(1, 256),
      w2, conv2_b.reshape(1, 512),
      fc0_w, fc0_b.reshape(1, 512), fc1_w, fc1_b.reshape(1, 512),
      fc2_w, fc2_b.reshape(1, class_num))
    return out


# speedup vs baseline: 1.2664x; 1.0131x over previous
import numpy as np

import jax
import jax.numpy as jnp
from jax.experimental import pallas as pl
from jax.experimental.pallas import tpu as pltpu

_VMEM_LIMIT = 100 * 1024 * 1024


def _conv0_onehot():
    s = np.zeros((3, 8, 8, 4, 16, 27), np.float32)
    for c in range(3):
        for iy in range(8):
            for ix in range(8):
                for py in range(2):
                    for px in range(2):
                        for oyp in range(4):
                            for oxp in range(4):
                                dy = iy - (2 * oyp + py) + 1
                                dx = ix - (2 * oxp + px) + 1
                                if 0 <= dy <= 2 and 0 <= dx <= 2:
                                    t = dy * 3 + dx
                                    s[c, iy, ix, py * 2 + px, oyp * 4 + oxp,
                                      t * 3 + c] = 1.0
    return s.reshape(192 * 64, 27)


_S0 = _conv0_onehot()


def _fused_vgg_kernel(x_ref, w0_ref, b0_ref, w1a_ref, w1b_ref, b1_ref,
                      w2_ref, b2_ref, f0w_ref, f0b_ref, f1w_ref, f1b_ref,
                      f2w_ref, f2b_ref, o_ref):
    B = x_ref.shape[0]
    f32 = jnp.float32
    bf16 = jnp.bfloat16

    def stream(xs):
        h = jnp.dot(xs.astype(bf16), w0_ref[...],
                    preferred_element_type=f32).astype(bf16)
        m = jnp.maximum(jnp.maximum(h[:, 0:2048], h[:, 2048:4096]),
                        jnp.maximum(h[:, 4096:6144], h[:, 6144:8192]))
        b0 = jnp.tile(b0_ref[...].astype(bf16), (1, 16))
        a1 = jnp.maximum(m + b0, 0.0)

        oy0 = jnp.dot(a1[:, 0:1024], w1a_ref[:, 0:1024],
                      preferred_element_type=f32).astype(bf16)
        oy1 = jnp.dot(a1[:, 0:1536], w1b_ref[:, 0:1024],
                      preferred_element_type=f32).astype(bf16)
        oy2 = jnp.dot(a1[:, 512:2048], w1b_ref[:, 1024:2048],
                      preferred_element_type=f32).astype(bf16)
        oy3 = jnp.dot(a1[:, 1024:2048], w1a_ref[:, 1024:2048],
                      preferred_element_type=f32).astype(bf16)
        r0 = jnp.maximum(jnp.maximum(oy0[:, 0:512], oy0[:, 512:1024]),
                         jnp.maximum(oy1[:, 0:512], oy1[:, 512:1024]))
        r1 = jnp.maximum(jnp.maximum(oy2[:, 0:512], oy2[:, 512:1024]),
                         jnp.maximum(oy3[:, 0:512], oy3[:, 512:1024]))
        b1 = jnp.tile(b1_ref[...].astype(bf16), (1, 2))
        a2a = jnp.maximum(r0 + b1, 0.0)
        a2b = jnp.maximum(r1 + b1, 0.0)
        a2 = jnp.concatenate([a2a, a2b], axis=1)

        g = jnp.dot(a2, w2_ref[...],
                    preferred_element_type=f32).astype(bf16)
        m2 = jnp.maximum(jnp.maximum(g[:, 0:512], g[:, 512:1024]),
                         jnp.maximum(g[:, 1024:1536], g[:, 1536:2048]))
        h = jnp.maximum(m2 + b2_ref[...].astype(bf16), 0.0)

        h = jnp.dot(h, f0w_ref[...], preferred_element_type=f32)
        h = jnp.maximum(h + f0b_ref[...], 0.0).astype(bf16)
        h = jnp.dot(h, f1w_ref[...], preferred_element_type=f32)
        h = jnp.maximum(h + f1b_ref[...], 0.0).astype(bf16)
        h = jnp.dot(h, f2w_ref[...], preferred_element_type=f32)
        return h + f2b_ref[...]

    Bs = 512 if B % 512 == 0 else B
    for s in range(B // Bs):
        o_ref[s * Bs:(s + 1) * Bs, :] = stream(x_ref[s * Bs:(s + 1) * Bs, :])


def _toeplitz_conv0(w9):
    s0 = jnp.asarray(_S0, jnp.bfloat16)
    w = jnp.dot(s0, w9.reshape(27, 128), preferred_element_type=jnp.float32)
    return w.reshape(192, 8192).astype(jnp.bfloat16)


def _conv1_row_mats(w9):
    z = jnp.zeros((128, 256), w9.dtype)
    ox_of_block = (0, 2, 1, 3)
    bdy = []
    for dy in range(3):
        rows = []
        for x in range(4):
            blocks = []
            for ox in ox_of_block:
                dx = x - ox + 1
                blocks.append(w9[dy * 3 + dx] if 0 <= dx <= 2 else z)
            rows.append(jnp.concatenate(blocks, axis=1))
        bdy.append(jnp.concatenate(rows, axis=0))
    b0, b1, b2 = bdy
    w1_oy0 = jnp.concatenate([b1, b2], axis=0)
    w1_oy1 = jnp.concatenate([b0, b1, b2], axis=0)
    w1_oy2 = jnp.concatenate([b0, b1, b2], axis=0)
    w1_oy3 = jnp.concatenate([b0, b1], axis=0)
    w1a = jnp.concatenate([w1_oy0, w1_oy3], axis=1).astype(jnp.bfloat16)
    w1b = jnp.concatenate([w1_oy1, w1_oy2], axis=1).astype(jnp.bfloat16)
    return w1a, w1b


def _toeplitz_conv2(w9):
    rows = []
    for iy in range(2):
        for ix in range(2):
            blocks = [w9[(iy - oy + 1) * 3 + (ix - ox + 1)]
                      for oy in range(2) for ox in range(2)]
            rows.append(jnp.concatenate(blocks, axis=1))
    return jnp.concatenate(rows, axis=0)


def kernel(x, conv0_w, conv0_b, conv1_w, conv1_b, conv2_w, conv2_b,
           fc0_w, fc0_b, fc1_w, fc1_b, fc2_w, fc2_b):
    n = x.shape[0]
    B = 1024 if n % 1024 == 0 else (128 if n % 128 == 0 else n)

    x2 = x.reshape(n, 192)

    w0 = _toeplitz_conv0(conv0_w)
    w1a, w1b = _conv1_row_mats(conv1_w)
    w2 = _toeplitz_conv2(conv2_w)
    class_num = fc2_w.shape[1]

    out = pl.pallas_call(
        _fused_vgg_kernel,
        grid=(n // B,),
        out_shape=jax.ShapeDtypeStruct((n, class_num), jnp.float32),
        in_specs=[
            pl.BlockSpec((B, 192), lambda i: (i, 0)),
            pl.BlockSpec((192, 8192), lambda i: (0, 0)),
            pl.BlockSpec((1, 128), lambda i: (0, 0)),
            pl.BlockSpec((1024, 2048), lambda i: (0, 0)),
            pl.BlockSpec((1536, 2048), lambda i: (0, 0)),
            pl.BlockSpec((1, 256), lambda i: (0, 0)),
            pl.BlockSpec((1024, 2048), lambda i: (0, 0)),
            pl.BlockSpec((1, 512), lambda i: (0, 0)),
            pl.BlockSpec((512, 512), lambda i: (0, 0)),
            pl.BlockSpec((1, 512), lambda i: (0, 0)),
            pl.BlockSpec((512, 512), lambda i: (0, 0)),
            pl.BlockSpec((1, 512), lambda i: (0, 0)),
            pl.BlockSpec((512, class_num), lambda i: (0, 0)),
            pl.BlockSpec((1, class_num), lambda i: (0, 0)),
        ],
        out_specs=pl.BlockSpec((B, class_num), lambda i: (i, 0)),
        compiler_params=pltpu.CompilerParams(
            dimension_semantics=("arbitrary",),
            vmem_limit_bytes=_VMEM_LIMIT,
        ),
    )(x2, w0, conv0_b.reshape(1, 128), w1a, w1b, conv1_b.reshape(1, 256),
      w2, conv2_b.reshape(1, 512),
      fc0_w, fc0_b.reshape(1, 512), fc1_w, fc1_b.reshape(1, 512),
      fc2_w, fc2_b.reshape(1, class_num))
    return out
